# Initial kernel scaffold; baseline (speedup 1.0000x reference)
import jax
import jax.numpy as jnp
from jax.experimental import pallas as pl


def kernel(edge_index, edge_weight, user_emb, item_emb):
    raise NotImplementedError("write your pallas kernel here")


# speedup vs baseline: 4.2525x; 4.2525x over previous
import dataclasses
import functools

import jax
import jax.numpy as jnp
from jax import lax
from jax.experimental import pallas as pl
from jax.experimental.pallas import tpu as pltpu
from jax.experimental.pallas import tpu_sc as plsc

D = 64
DH = 32
NPAD = 50176
STRIP = NPAD // 16
PIECE = 448
SUP = 49
SUB = 8
CHUNK = 128
EPAD = 16 * SUP * SUB * CHUNK


def _make_layer():
    scratch = [
        pltpu.VMEM((SUB, CHUNK), jnp.int32),
        pltpu.VMEM((SUB, CHUNK), jnp.int32),
        pltpu.VMEM((SUB, CHUNK), jnp.float32),
        pltpu.VMEM((CHUNK, DH), jnp.float32),
        pltpu.VMEM((PIECE, DH), jnp.float32),
        pltpu.VMEM_SHARED((NPAD, DH), jnp.float32),
        pltpu.SemaphoreType.DMA,
    ]
    out_t = jax.ShapeDtypeStruct((2 * NPAD, DH), jnp.float32)
    mesh = plsc.VectorSubcoreMesh(core_axis_name="c", subcore_axis_name="s")

    def body(emb, rowh, colh, wh, out, colb, rowb, wb, rows, piece, acc, sem):
        c = lax.axis_index("c")
        s = lax.axis_index("s")
        zero16 = jnp.zeros((16,), jnp.float32)
        tabbase = c * NPAD

        @pl.loop(0, PIECE)
        def _(i):
            for q in range(DH // 16):
                piece[i, pl.ds(q * 16, 16)] = zero16
        for p in range(STRIP // PIECE):
            pltpu.sync_copy(piece, acc.at[pl.ds(s * STRIP + p * PIECE, PIECE)])
        plsc.subcore_barrier()

        @pl.loop(0, SUP)
        def _(g):
            rbase = (s * SUP + g) * SUB
            pltpu.sync_copy(rowh.at[pl.ds(rbase, SUB)], rowb)
            pltpu.sync_copy(colh.at[pl.ds(rbase, SUB)], colb)
            pltpu.sync_copy(wh.at[pl.ds(rbase, SUB)], wb)
            for jj in range(SUB):
                for k in range(SUB):
                    sl = pl.ds(k * 16, 16)
                    colb[jj, sl] = colb[jj, sl] + tabbase
            for jj in range(SUB):
                pltpu.async_copy(emb.at[colb.at[jj]], rows, sem).wait()

                @pl.loop(0, CHUNK)
                def _(e):
                    wsp = plsc.load_gather(
                        wb, [jnp.full((16,), jj, jnp.int32),
                             jnp.full((16,), e, jnp.int32)])
                    for q in range(DH // 16):
                        sl = pl.ds(q * 16, 16)
                        rows[e, sl] = rows[e, sl] * wsp

                pltpu.sync_copy(rows, acc.at[rowb.at[jj]], add=True)
        plsc.subcore_barrier()

        for p in range(STRIP // PIECE):
            lo = s * STRIP + p * PIECE
            pltpu.sync_copy(acc.at[pl.ds(lo, PIECE)], piece)
            pltpu.sync_copy(piece, out.at[pl.ds(tabbase + lo, PIECE)])

    cp = pltpu.CompilerParams(use_tc_tiling_on_sc=False)
    if "needs_layout_passes" in pltpu.CompilerParams.__dataclass_fields__:
        cp = dataclasses.replace(cp, needs_layout_passes=False)
    return functools.partial(
        pl.kernel, out_type=out_t, mesh=mesh, scratch_types=scratch,
        compiler_params=cp)(body)


_layer = _make_layer()

_BM = 512


def _mean_body(e0, e1l, e1r, e2l, e2r, out):
    third = jnp.float32(1.0 / 3.0)
    out[:, :DH] = (e0[:, :DH] + e1l[...] + e2l[...]) * third
    out[:, DH:] = (e0[:, DH:] + e1r[...] + e2r[...]) * third


def _mean(e0, e1, e2):
    nb = NPAD // _BM
    half = lambda off: pl.BlockSpec((_BM, DH), lambda i, o=off: (i + o, 0))
    return pl.pallas_call(
        _mean_body,
        grid=(nb,),
        in_specs=[pl.BlockSpec((_BM, D), lambda i: (i, 0)),
                  half(0), half(nb), half(0), half(nb)],
        out_specs=pl.BlockSpec((_BM, D), lambda i: (i, 0)),
        out_shape=jax.ShapeDtypeStruct((NPAD, D), jnp.float32),
    )(e0, e1, e1, e2, e2)


def kernel(edge_index, edge_weight, user_emb, item_emb):
    row = edge_index[0]
    col = edge_index[1]
    e = row.shape[0]
    padn = EPAD - e
    pad_i = jnp.zeros((padn,), jnp.int32)
    rowp = jnp.concatenate([row, pad_i]).reshape(EPAD // CHUNK, CHUNK)
    colp = jnp.concatenate([col, pad_i]).reshape(EPAD // CHUNK, CHUNK)
    wp = jnp.concatenate(
        [edge_weight, jnp.zeros((padn,), jnp.float32)]
    ).reshape(EPAD // CHUNK, CHUNK)
    nu = user_emb.shape[0]
    ni = item_emb.shape[0]
    emb0 = jnp.concatenate(
        [user_emb, item_emb, jnp.zeros((NPAD - nu - ni, D), jnp.float32)],
        axis=0)
    emb0s = jnp.concatenate([emb0[:, :DH], emb0[:, DH:]], axis=0)
    emb1s = _layer(emb0s, rowp, colp, wp)
    emb2s = _layer(emb1s, rowp, colp, wp)
    fin = _mean(emb0, emb1s, emb2s)
    return fin[:nu], fin[nu:nu + ni]


# speedup vs baseline: 5.8349x; 1.3721x over previous
import dataclasses
import functools

import jax
import jax.numpy as jnp
from jax import lax
from jax.experimental import pallas as pl
from jax.experimental.pallas import tpu as pltpu
from jax.experimental.pallas import tpu_sc as plsc

D = 64
DH = 32
NPAD = 50176
STRIP = NPAD // 16
PIECE = 448
SUP = 49
SUB = 8
CHUNK = 128
EPAD = 16 * SUP * SUB * CHUNK


def _make_layer():
    scratch = [
        pltpu.VMEM((SUB, CHUNK), jnp.int32),
        pltpu.VMEM((SUB, CHUNK), jnp.int32),
        pltpu.VMEM((SUB, CHUNK), jnp.float32),
        pltpu.VMEM((CHUNK, DH), jnp.float32),
        pltpu.VMEM((CHUNK, DH), jnp.float32),
        pltpu.VMEM((PIECE, DH), jnp.float32),
        pltpu.VMEM_SHARED((NPAD, DH), jnp.float32),
        pltpu.SemaphoreType.DMA,
        pltpu.SemaphoreType.DMA,
    ]
    out_t = jax.ShapeDtypeStruct((2 * NPAD, DH), jnp.float32)
    mesh = plsc.VectorSubcoreMesh(core_axis_name="c", subcore_axis_name="s")

    def body(emb, rowh, colh, wh, out, colb, rowb, wb, rows0, rows1, piece,
             acc, gsem0, gsem1):
        c = lax.axis_index("c")
        s = lax.axis_index("s")
        zero16 = jnp.zeros((16,), jnp.float32)
        bufs = (rows0, rows1)
        gsems = (gsem0, gsem1)
        tabbase = c * NPAD

        @pl.loop(0, PIECE)
        def _(i):
            for q in range(DH // 16):
                piece[i, pl.ds(q * 16, 16)] = zero16
        for p in range(STRIP // PIECE):
            pltpu.sync_copy(piece, acc.at[pl.ds(s * STRIP + p * PIECE, PIECE)])
        plsc.subcore_barrier()

        def scale(buf, jj):
            @pl.loop(0, CHUNK, unroll=4)
            def _(e):
                wsp = plsc.load_gather(
                    wb, [jnp.full((16,), jj, jnp.int32),
                         jnp.full((16,), e, jnp.int32)])
                for q in range(DH // 16):
                    sl = pl.ds(q * 16, 16)
                    buf[e, sl] = buf[e, sl] * wsp

        @pl.loop(0, SUP)
        def _(g):
            rbase = (s * SUP + g) * SUB
            pltpu.sync_copy(rowh.at[pl.ds(rbase, SUB)], rowb)
            pltpu.sync_copy(colh.at[pl.ds(rbase, SUB)], colb)
            pltpu.sync_copy(wh.at[pl.ds(rbase, SUB)], wb)
            for jj in range(SUB):
                for k in range(SUB):
                    sl = pl.ds(k * 16, 16)
                    colb[jj, sl] = colb[jj, sl] + tabbase
            gd = [None, None]
            gd[0] = pltpu.async_copy(emb.at[colb.at[0]], bufs[0], gsems[0])
            for jj in range(SUB):
                b = jj % 2
                nb = 1 - b
                gd[b].wait()
                if jj + 1 < SUB:
                    gd[nb] = pltpu.async_copy(
                        emb.at[colb.at[jj + 1]], bufs[nb], gsems[nb])
                scale(bufs[b], jj)
                pltpu.sync_copy(bufs[b], acc.at[rowb.at[jj]], add=True)
        plsc.subcore_barrier()

        for p in range(STRIP // PIECE):
            lo = s * STRIP + p * PIECE
            pltpu.sync_copy(acc.at[pl.ds(lo, PIECE)], piece)
            pltpu.sync_copy(piece, out.at[pl.ds(tabbase + lo, PIECE)])

    cp = pltpu.CompilerParams(use_tc_tiling_on_sc=False)
    if "needs_layout_passes" in pltpu.CompilerParams.__dataclass_fields__:
        cp = dataclasses.replace(cp, needs_layout_passes=False)
    return functools.partial(
        pl.kernel, out_type=out_t, mesh=mesh, scratch_types=scratch,
        compiler_params=cp)(body)


_layer = _make_layer()

_BM = 512


def _mean_body(e0, e1l, e1r, e2l, e2r, out):
    third = jnp.float32(1.0 / 3.0)
    out[:, :DH] = (e0[:, :DH] + e1l[...] + e2l[...]) * third
    out[:, DH:] = (e0[:, DH:] + e1r[...] + e2r[...]) * third


def _mean(e0, e1, e2):
    nb = NPAD // _BM
    half = lambda h: pl.BlockSpec((_BM, DH), lambda i, h=h: (i + h * nb, 0))
    return pl.pallas_call(
        _mean_body,
        grid=(nb,),
        in_specs=[pl.BlockSpec((_BM, D), lambda i: (i, 0)),
                  half(0), half(1), half(0), half(1)],
        out_specs=pl.BlockSpec((_BM, D), lambda i: (i, 0)),
        out_shape=jax.ShapeDtypeStruct((NPAD, D), jnp.float32),
    )(e0, e1, e1, e2, e2)


def kernel(edge_index, edge_weight, user_emb, item_emb):
    row = edge_index[0]
    col = edge_index[1]
    e = row.shape[0]
    padn = EPAD - e
    pad_i = jnp.zeros((padn,), jnp.int32)
    rowp = jnp.concatenate([row, pad_i]).reshape(EPAD // CHUNK, CHUNK)
    colp = jnp.concatenate([col, pad_i]).reshape(EPAD // CHUNK, CHUNK)
    wp = jnp.concatenate(
        [edge_weight, jnp.zeros((padn,), jnp.float32)]
    ).reshape(EPAD // CHUNK, CHUNK)
    nu = user_emb.shape[0]
    ni = item_emb.shape[0]
    emb0 = jnp.concatenate(
        [user_emb, item_emb, jnp.zeros((NPAD - nu - ni, D), jnp.float32)],
        axis=0)
    emb0s = jnp.concatenate([emb0[:, :DH], emb0[:, DH:]], axis=0)
    emb1s = _layer(emb0s, rowp, colp, wp)
    emb2s = _layer(emb1s, rowp, colp, wp)
    fin = _mean(emb0, emb1s, emb2s)
    return fin[:nu], fin[nu:nu + ni]


# speedup vs baseline: 5.9104x; 1.0129x over previous
import dataclasses
import functools

import jax
import jax.numpy as jnp
from jax import lax
from jax.experimental import pallas as pl
from jax.experimental.pallas import tpu as pltpu
from jax.experimental.pallas import tpu_sc as plsc

D = 64
DH = 32
NPAD = 50176
STRIP = NPAD // 16
PIECE = 448
SUP = 49
SUB = 8
CHUNK = 128
EPAD = 16 * SUP * SUB * CHUNK


def _make_layer():
    scratch = [
        pltpu.VMEM((SUB, CHUNK), jnp.int32),
        pltpu.VMEM((SUB, CHUNK), jnp.int32),
        pltpu.VMEM((SUB, CHUNK), jnp.float32),
        pltpu.VMEM((CHUNK, DH), jnp.float32),
        pltpu.VMEM((CHUNK, DH), jnp.float32),
        pltpu.VMEM((PIECE, DH), jnp.float32),
        pltpu.VMEM_SHARED((NPAD, DH), jnp.float32),
        pltpu.SemaphoreType.DMA,
        pltpu.SemaphoreType.DMA,
        pltpu.SemaphoreType.DMA,
        pltpu.SemaphoreType.DMA,
    ]
    out_t = jax.ShapeDtypeStruct((2 * NPAD, DH), jnp.float32)
    mesh = plsc.VectorSubcoreMesh(core_axis_name="c", subcore_axis_name="s")

    def body(emb, rowh, colh, wh, out, colb, rowb, wb, rows0, rows1, piece,
             acc, gsem0, gsem1, ssem0, ssem1):
        c = lax.axis_index("c")
        s = lax.axis_index("s")
        zero16 = jnp.zeros((16,), jnp.float32)
        bufs = (rows0, rows1)
        gsems = (gsem0, gsem1)
        ssems = (ssem0, ssem1)
        tabbase = c * NPAD

        @pl.loop(0, PIECE)
        def _(i):
            for q in range(DH // 16):
                piece[i, pl.ds(q * 16, 16)] = zero16
        for p in range(STRIP // PIECE):
            pltpu.sync_copy(piece, acc.at[pl.ds(s * STRIP + p * PIECE, PIECE)])
        plsc.subcore_barrier()

        def scale(buf, jj):
            @pl.loop(0, CHUNK, unroll=4)
            def _(e):
                wsp = plsc.load_gather(
                    wb, [jnp.full((16,), jj, jnp.int32),
                         jnp.full((16,), e, jnp.int32)])
                for q in range(DH // 16):
                    sl = pl.ds(q * 16, 16)
                    buf[e, sl] = buf[e, sl] * wsp

        @pl.loop(0, SUP)
        def _(g):
            rbase = (s * SUP + g) * SUB
            pltpu.sync_copy(rowh.at[pl.ds(rbase, SUB)], rowb)
            pltpu.sync_copy(colh.at[pl.ds(rbase, SUB)], colb)
            pltpu.sync_copy(wh.at[pl.ds(rbase, SUB)], wb)
            for jj in range(SUB):
                for k in range(SUB):
                    sl = pl.ds(k * 16, 16)
                    colb[jj, sl] = colb[jj, sl] + tabbase
            gd = [None, None]
            sd = [None, None]

            def wait_sd(x):
                if sd[x] is not None:
                    sd[x].wait()
                    sd[x] = None

            gd[0] = pltpu.async_copy(emb.at[colb.at[0]], bufs[0], gsems[0])
            for jj in range(SUB):
                b = jj % 2
                nb = 1 - b
                gd[b].wait()
                if jj + 1 < SUB:
                    wait_sd(nb)
                    gd[nb] = pltpu.async_copy(
                        emb.at[colb.at[jj + 1]], bufs[nb], gsems[nb])
                scale(bufs[b], jj)
                sd[b] = pltpu.async_copy(bufs[b], acc.at[rowb.at[jj]],
                                         ssems[b], add=True)
            wait_sd(0)
            wait_sd(1)
        plsc.subcore_barrier()

        for p in range(STRIP // PIECE):
            lo = s * STRIP + p * PIECE
            pltpu.sync_copy(acc.at[pl.ds(lo, PIECE)], piece)
            pltpu.sync_copy(piece, out.at[pl.ds(tabbase + lo, PIECE)])

    cp = pltpu.CompilerParams(use_tc_tiling_on_sc=False)
    if "needs_layout_passes" in pltpu.CompilerParams.__dataclass_fields__:
        cp = dataclasses.replace(cp, needs_layout_passes=False)
    return functools.partial(
        pl.kernel, out_type=out_t, mesh=mesh, scratch_types=scratch,
        compiler_params=cp)(body)


_layer = _make_layer()

_BM = 512


def _mean_body(e0, e1l, e1r, e2l, e2r, out):
    third = jnp.float32(1.0 / 3.0)
    out[:, :DH] = (e0[:, :DH] + e1l[...] + e2l[...]) * third
    out[:, DH:] = (e0[:, DH:] + e1r[...] + e2r[...]) * third


def _mean(e0, e1, e2):
    nb = NPAD // _BM
    half = lambda h: pl.BlockSpec((_BM, DH), lambda i, h=h: (i + h * nb, 0))
    return pl.pallas_call(
        _mean_body,
        grid=(nb,),
        in_specs=[pl.BlockSpec((_BM, D), lambda i: (i, 0)),
                  half(0), half(1), half(0), half(1)],
        out_specs=pl.BlockSpec((_BM, D), lambda i: (i, 0)),
        out_shape=jax.ShapeDtypeStruct((NPAD, D), jnp.float32),
    )(e0, e1, e1, e2, e2)


def kernel(edge_index, edge_weight, user_emb, item_emb):
    row = edge_index[0]
    col = edge_index[1]
    e = row.shape[0]
    padn = EPAD - e
    pad_i = jnp.zeros((padn,), jnp.int32)
    rowp = jnp.concatenate([row, pad_i]).reshape(EPAD // CHUNK, CHUNK)
    colp = jnp.concatenate([col, pad_i]).reshape(EPAD // CHUNK, CHUNK)
    wp = jnp.concatenate(
        [edge_weight, jnp.zeros((padn,), jnp.float32)]
    ).reshape(EPAD // CHUNK, CHUNK)
    nu = user_emb.shape[0]
    ni = item_emb.shape[0]
    emb0 = jnp.concatenate(
        [user_emb, item_emb, jnp.zeros((NPAD - nu - ni, D), jnp.float32)],
        axis=0)
    emb0s = jnp.concatenate([emb0[:, :DH], emb0[:, DH:]], axis=0)
    emb1s = _layer(emb0s, rowp, colp, wp)
    emb2s = _layer(emb1s, rowp, colp, wp)
    fin = _mean(emb0, emb1s, emb2s)
    return fin[:nu], fin[nu:nu + ni]


# speedup vs baseline: 6.6540x; 1.1258x over previous
import dataclasses
import functools

import jax
import jax.numpy as jnp
from jax import lax
from jax.experimental import pallas as pl
from jax.experimental.pallas import tpu as pltpu
from jax.experimental.pallas import tpu_sc as plsc

D = 64
DH = 32
NPAD = 50176
STRIP = NPAD // 16
PIECE = 448
SUP = 49
SUB = 8
CHUNK = 128
EPAD = 16 * SUP * SUB * CHUNK


def _make_layer():
    scratch = [
        pltpu.VMEM((SUB, CHUNK), jnp.int32),
        pltpu.VMEM((SUB, CHUNK), jnp.int32),
        pltpu.VMEM((SUB, CHUNK), jnp.float32),
        pltpu.VMEM((CHUNK, DH), jnp.float32),
        pltpu.VMEM((CHUNK, DH), jnp.float32),
        pltpu.VMEM((PIECE, DH), jnp.float32),
        pltpu.VMEM_SHARED((NPAD, DH), jnp.float32),
        pltpu.SemaphoreType.DMA,
        pltpu.SemaphoreType.DMA,
        pltpu.SemaphoreType.DMA,
        pltpu.SemaphoreType.DMA,
    ]
    out_t = jax.ShapeDtypeStruct((2 * NPAD, DH), jnp.float32)
    mesh = plsc.VectorSubcoreMesh(core_axis_name="c", subcore_axis_name="s")

    def body(emb, rowh, colh, wh, out, colb, rowb, wb, rows0, rows1, piece,
             acc, gsem0, gsem1, ssem0, ssem1):
        c = lax.axis_index("c")
        s = lax.axis_index("s")
        zero16 = jnp.zeros((16,), jnp.float32)
        bufs = (rows0, rows1)
        gsems = (gsem0, gsem1)
        ssems = (ssem0, ssem1)
        tabbase = c * NPAD

        @pl.loop(0, PIECE)
        def _(i):
            for q in range(DH // 16):
                piece[i, pl.ds(q * 16, 16)] = zero16
        for p in range(STRIP // PIECE):
            pltpu.sync_copy(piece, acc.at[pl.ds(s * STRIP + p * PIECE, PIECE)])
        plsc.subcore_barrier()

        def scale(buf, jj):
            @pl.loop(0, CHUNK // 16)
            def _(g16):
                w16 = wb[jj, pl.ds(g16 * 16, 16)]
                base = g16 * 16
                for e in range(16):
                    wsp = jnp.full((16,), w16[e])
                    for q in range(DH // 16):
                        sl = pl.ds(q * 16, 16)
                        buf[base + e, sl] = buf[base + e, sl] * wsp

        @pl.loop(0, SUP)
        def _(g):
            rbase = (s * SUP + g) * SUB
            pltpu.sync_copy(rowh.at[pl.ds(rbase, SUB)], rowb)
            pltpu.sync_copy(colh.at[pl.ds(rbase, SUB)], colb)
            pltpu.sync_copy(wh.at[pl.ds(rbase, SUB)], wb)
            for jj in range(SUB):
                for k in range(SUB):
                    sl = pl.ds(k * 16, 16)
                    colb[jj, sl] = colb[jj, sl] + tabbase
            gd = [None, None]
            sd = [None, None]

            def wait_sd(x):
                if sd[x] is not None:
                    sd[x].wait()
                    sd[x] = None

            gd[0] = pltpu.async_copy(emb.at[colb.at[0]], bufs[0], gsems[0])
            for jj in range(SUB):
                b = jj % 2
                nb = 1 - b
                gd[b].wait()
                if jj + 1 < SUB:
                    wait_sd(nb)
                    gd[nb] = pltpu.async_copy(
                        emb.at[colb.at[jj + 1]], bufs[nb], gsems[nb])
                scale(bufs[b], jj)
                sd[b] = pltpu.async_copy(bufs[b], acc.at[rowb.at[jj]],
                                         ssems[b], add=True)
            wait_sd(0)
            wait_sd(1)
        plsc.subcore_barrier()

        for p in range(STRIP // PIECE):
            lo = s * STRIP + p * PIECE
            pltpu.sync_copy(acc.at[pl.ds(lo, PIECE)], piece)
            pltpu.sync_copy(piece, out.at[pl.ds(tabbase + lo, PIECE)])

    cp = pltpu.CompilerParams(use_tc_tiling_on_sc=False)
    if "needs_layout_passes" in pltpu.CompilerParams.__dataclass_fields__:
        cp = dataclasses.replace(cp, needs_layout_passes=False)
    return functools.partial(
        pl.kernel, out_type=out_t, mesh=mesh, scratch_types=scratch,
        compiler_params=cp)(body)


_layer = _make_layer()

_BM = 512


def _mean_body(e0, e1l, e1r, e2l, e2r, out):
    third = jnp.float32(1.0 / 3.0)
    out[:, :DH] = (e0[:, :DH] + e1l[...] + e2l[...]) * third
    out[:, DH:] = (e0[:, DH:] + e1r[...] + e2r[...]) * third


def _mean(e0, e1, e2):
    nb = NPAD // _BM
    half = lambda h: pl.BlockSpec((_BM, DH), lambda i, h=h: (i + h * nb, 0))
    return pl.pallas_call(
        _mean_body,
        grid=(nb,),
        in_specs=[pl.BlockSpec((_BM, D), lambda i: (i, 0)),
                  half(0), half(1), half(0), half(1)],
        out_specs=pl.BlockSpec((_BM, D), lambda i: (i, 0)),
        out_shape=jax.ShapeDtypeStruct((NPAD, D), jnp.float32),
    )(e0, e1, e1, e2, e2)


def kernel(edge_index, edge_weight, user_emb, item_emb):
    row = edge_index[0]
    col = edge_index[1]
    e = row.shape[0]
    padn = EPAD - e
    pad_i = jnp.zeros((padn,), jnp.int32)
    rowp = jnp.concatenate([row, pad_i]).reshape(EPAD // CHUNK, CHUNK)
    colp = jnp.concatenate([col, pad_i]).reshape(EPAD // CHUNK, CHUNK)
    wp = jnp.concatenate(
        [edge_weight, jnp.zeros((padn,), jnp.float32)]
    ).reshape(EPAD // CHUNK, CHUNK)
    nu = user_emb.shape[0]
    ni = item_emb.shape[0]
    emb0 = jnp.concatenate(
        [user_emb, item_emb, jnp.zeros((NPAD - nu - ni, D), jnp.float32)],
        axis=0)
    emb0s = jnp.concatenate([emb0[:, :DH], emb0[:, DH:]], axis=0)
    emb1s = _layer(emb0s, rowp, colp, wp)
    emb2s = _layer(emb1s, rowp, colp, wp)
    fin = _mean(emb0, emb1s, emb2s)
    return fin[:nu], fin[nu:nu + ni]


# speedup vs baseline: 7.1316x; 1.0718x over previous
import dataclasses
import functools

import jax
import jax.numpy as jnp
from jax import lax
from jax.experimental import pallas as pl
from jax.experimental.pallas import tpu as pltpu
from jax.experimental.pallas import tpu_sc as plsc

D = 64
DH = 32
NPAD = 50176
STRIP = NPAD // 16
PIECE = 448
SUP = 28
SUB = 14
CHUNK = 128
EPAD = 16 * SUP * SUB * CHUNK


def _make_layer():
    scratch = [
        pltpu.VMEM((3, SUB, CHUNK), jnp.int32),
        pltpu.VMEM((CHUNK, DH), jnp.float32),
        pltpu.VMEM((CHUNK, DH), jnp.float32),
        pltpu.VMEM((PIECE, DH), jnp.float32),
        pltpu.VMEM_SHARED((NPAD, DH), jnp.float32),
        pltpu.SemaphoreType.DMA,
        pltpu.SemaphoreType.DMA,
        pltpu.SemaphoreType.DMA,
        pltpu.SemaphoreType.DMA,
    ]
    out_t = jax.ShapeDtypeStruct((2 * NPAD, DH), jnp.float32)
    mesh = plsc.VectorSubcoreMesh(core_axis_name="c", subcore_axis_name="s")

    def body(emb, edg, out, eb, rows0, rows1, piece,
             acc, gsem0, gsem1, ssem0, ssem1):
        c = lax.axis_index("c")
        s = lax.axis_index("s")
        zero16 = jnp.zeros((16,), jnp.float32)
        bufs = (rows0, rows1)
        gsems = (gsem0, gsem1)
        ssems = (ssem0, ssem1)
        tabbase = c * NPAD

        @pl.loop(0, PIECE)
        def _(i):
            for q in range(DH // 16):
                piece[i, pl.ds(q * 16, 16)] = zero16
        for p in range(STRIP // PIECE):
            pltpu.sync_copy(piece, acc.at[pl.ds(s * STRIP + p * PIECE, PIECE)])
        plsc.subcore_barrier()

        def scale(buf, jj):
            @pl.loop(0, CHUNK // 16)
            def _(g16):
                w16 = plsc.bitcast(eb[2, jj, pl.ds(g16 * 16, 16)], jnp.float32)
                base = g16 * 16
                for e in range(16):
                    wsp = jnp.full((16,), w16[e])
                    for q in range(DH // 16):
                        sl = pl.ds(q * 16, 16)
                        buf[base + e, sl] = buf[base + e, sl] * wsp

        @pl.loop(0, SUP)
        def _(g):
            pltpu.sync_copy(edg.at[s * SUP + g], eb)
            for jj in range(SUB):
                for k in range(CHUNK // 16):
                    sl = pl.ds(k * 16, 16)
                    eb[1, jj, sl] = eb[1, jj, sl] + tabbase
            gd = [None, None]
            sd = [None, None]

            def wait_sd(x):
                if sd[x] is not None:
                    sd[x].wait()
                    sd[x] = None

            gd[0] = pltpu.async_copy(emb.at[eb.at[1].at[0]], bufs[0], gsems[0])
            for jj in range(SUB):
                b = jj % 2
                nb = 1 - b
                gd[b].wait()
                if jj + 1 < SUB:
                    wait_sd(nb)
                    gd[nb] = pltpu.async_copy(
                        emb.at[eb.at[1].at[jj + 1]], bufs[nb], gsems[nb])
                scale(bufs[b], jj)
                sd[b] = pltpu.async_copy(bufs[b], acc.at[eb.at[0].at[jj]],
                                         ssems[b], add=True)
            wait_sd(0)
            wait_sd(1)
        plsc.subcore_barrier()

        for p in range(STRIP // PIECE):
            lo = s * STRIP + p * PIECE
            pltpu.sync_copy(acc.at[pl.ds(lo, PIECE)], piece)
            pltpu.sync_copy(piece, out.at[pl.ds(tabbase + lo, PIECE)])

    cp = pltpu.CompilerParams(use_tc_tiling_on_sc=False)
    if "needs_layout_passes" in pltpu.CompilerParams.__dataclass_fields__:
        cp = dataclasses.replace(cp, needs_layout_passes=False)
    return functools.partial(
        pl.kernel, out_type=out_t, mesh=mesh, scratch_types=scratch,
        compiler_params=cp)(body)


_layer = _make_layer()

_BM = 512


def _mean_body(e0, e1l, e1r, e2l, e2r, out):
    third = jnp.float32(1.0 / 3.0)
    out[:, :DH] = (e0[:, :DH] + e1l[...] + e2l[...]) * third
    out[:, DH:] = (e0[:, DH:] + e1r[...] + e2r[...]) * third


def _mean(e0, e1, e2):
    nb = NPAD // _BM
    half = lambda h: pl.BlockSpec((_BM, DH), lambda i, h=h: (i + h * nb, 0))
    return pl.pallas_call(
        _mean_body,
        grid=(nb,),
        in_specs=[pl.BlockSpec((_BM, D), lambda i: (i, 0)),
                  half(0), half(1), half(0), half(1)],
        out_specs=pl.BlockSpec((_BM, D), lambda i: (i, 0)),
        out_shape=jax.ShapeDtypeStruct((NPAD, D), jnp.float32),
    )(e0, e1, e1, e2, e2)


def kernel(edge_index, edge_weight, user_emb, item_emb):
    row = edge_index[0]
    col = edge_index[1]
    e = row.shape[0]
    padn = EPAD - e
    pad_i = jnp.zeros((padn,), jnp.int32)
    nsuper = 16 * SUP
    shp = (nsuper, SUB, CHUNK)
    rowp = jnp.concatenate([row, pad_i]).reshape(shp)
    colp = jnp.concatenate([col, pad_i]).reshape(shp)
    wbits = lax.bitcast_convert_type(
        jnp.concatenate([edge_weight, jnp.zeros((padn,), jnp.float32)]),
        jnp.int32).reshape(shp)
    edg = jnp.stack([rowp, colp, wbits], axis=1)
    nu = user_emb.shape[0]
    ni = item_emb.shape[0]
    emb0 = jnp.concatenate(
        [user_emb, item_emb, jnp.zeros((NPAD - nu - ni, D), jnp.float32)],
        axis=0)
    emb0s = jnp.concatenate([emb0[:, :DH], emb0[:, DH:]], axis=0)
    emb1s = _layer(emb0s, edg)
    emb2s = _layer(emb1s, edg)
    fin = _mean(emb0, emb1s, emb2s)
    return fin[:nu], fin[nu:nu + ni]


# speedup vs baseline: 8.5926x; 1.2049x over previous
import dataclasses
import functools

import jax
import jax.numpy as jnp
from jax import lax
from jax.experimental import pallas as pl
from jax.experimental.pallas import tpu as pltpu
from jax.experimental.pallas import tpu_sc as plsc

D = 64
DH = 32
NPAD = 50176
STRIP = NPAD // 16
PIECE = 224
SUP = 28
SUB = 14
CHUNK = 128
EPAD = 16 * SUP * SUB * CHUNK
NRING = 4
LOOKAHEAD = 2


def _make_layer():
    scratch = (
        [pltpu.VMEM((3, SUB, CHUNK), jnp.int32)]
        + [pltpu.VMEM((CHUNK, DH), jnp.float32)
           for _ in range(NRING)]
        + [pltpu.VMEM((PIECE, DH), jnp.float32)]
        + [pltpu.VMEM_SHARED((NPAD, DH), jnp.float32)]
        + [pltpu.SemaphoreType.DMA for _ in range(2 * NRING)]
    )
    out_t = jax.ShapeDtypeStruct((2 * NPAD, DH), jnp.float32)
    mesh = plsc.VectorSubcoreMesh(core_axis_name="c", subcore_axis_name="s")

    def body(emb, edg, out, eb, *rest):
        bufs = rest[:NRING]
        piece = rest[NRING]
        acc = rest[NRING + 1]
        gsems = rest[NRING + 2:2 * NRING + 2]
        ssems = rest[2 * NRING + 2:]
        c = lax.axis_index("c")
        s = lax.axis_index("s")
        zero16 = jnp.zeros((16,), jnp.float32)
        tabbase = c * NPAD

        @pl.loop(0, PIECE)
        def _(i):
            for q in range(DH // 16):
                piece[i, pl.ds(q * 16, 16)] = zero16
        for p in range(STRIP // PIECE):
            pltpu.sync_copy(piece, acc.at[pl.ds(s * STRIP + p * PIECE, PIECE)])
        plsc.subcore_barrier()

        def scale(buf, jj):
            @pl.loop(0, CHUNK // 16)
            def _(g16):
                w16 = plsc.bitcast(eb[2, jj, pl.ds(g16 * 16, 16)], jnp.float32)
                base = g16 * 16
                for e in range(16):
                    wsp = jnp.full((16,), w16[e])
                    for q in range(DH // 16):
                        sl = pl.ds(q * 16, 16)
                        buf[base + e, sl] = buf[base + e, sl] * wsp

        @pl.loop(0, SUP)
        def _(g):
            pltpu.sync_copy(edg.at[s * SUP + g], eb)
            for jj in range(SUB):
                for k in range(CHUNK // 16):
                    sl = pl.ds(k * 16, 16)
                    eb[1, jj, sl] = eb[1, jj, sl] + tabbase
            gd = [None] * NRING
            sd = [None] * NRING

            def wait_sd(x):
                if sd[x] is not None:
                    sd[x].wait()
                    sd[x] = None

            def start_gather(jx):
                bx = jx % NRING
                wait_sd(bx)
                gd[bx] = pltpu.async_copy(
                    emb.at[eb.at[1].at[jx]], bufs[bx], gsems[bx])

            for jx in range(min(LOOKAHEAD, SUB)):
                start_gather(jx)
            for jj in range(SUB):
                b = jj % NRING
                gd[b].wait()
                scale(bufs[b], jj)
                sd[b] = pltpu.async_copy(bufs[b], acc.at[eb.at[0].at[jj]],
                                         ssems[b], add=True)
                if jj + LOOKAHEAD < SUB:
                    start_gather(jj + LOOKAHEAD)
            for x in range(NRING):
                wait_sd(x)
        plsc.subcore_barrier()

        for p in range(STRIP // PIECE):
            lo = s * STRIP + p * PIECE
            pltpu.sync_copy(acc.at[pl.ds(lo, PIECE)], piece)
            pltpu.sync_copy(piece, out.at[pl.ds(tabbase + lo, PIECE)])

    cp = pltpu.CompilerParams(use_tc_tiling_on_sc=False)
    if "needs_layout_passes" in pltpu.CompilerParams.__dataclass_fields__:
        cp = dataclasses.replace(cp, needs_layout_passes=False)
    return functools.partial(
        pl.kernel, out_type=out_t, mesh=mesh, scratch_types=scratch,
        compiler_params=cp)(body)


_layer = _make_layer()

_BM = 512


def _mean_body(e0, e1l, e1r, e2l, e2r, out):
    third = jnp.float32(1.0 / 3.0)
    out[:, :DH] = (e0[:, :DH] + e1l[...] + e2l[...]) * third
    out[:, DH:] = (e0[:, DH:] + e1r[...] + e2r[...]) * third


def _mean(e0, e1, e2):
    nb = NPAD // _BM
    half = lambda h: pl.BlockSpec((_BM, DH), lambda i, h=h: (i + h * nb, 0))
    return pl.pallas_call(
        _mean_body,
        grid=(nb,),
        in_specs=[pl.BlockSpec((_BM, D), lambda i: (i, 0)),
                  half(0), half(1), half(0), half(1)],
        out_specs=pl.BlockSpec((_BM, D), lambda i: (i, 0)),
        out_shape=jax.ShapeDtypeStruct((NPAD, D), jnp.float32),
    )(e0, e1, e1, e2, e2)


def kernel(edge_index, edge_weight, user_emb, item_emb):
    row = edge_index[0]
    col = edge_index[1]
    e = row.shape[0]
    padn = EPAD - e
    pad_i = jnp.zeros((padn,), jnp.int32)
    nsuper = 16 * SUP
    shp = (nsuper, SUB, CHUNK)
    rowp = jnp.concatenate([row, pad_i]).reshape(shp)
    colp = jnp.concatenate([col, pad_i]).reshape(shp)
    wbits = lax.bitcast_convert_type(
        jnp.concatenate([edge_weight, jnp.zeros((padn,), jnp.float32)]),
        jnp.int32).reshape(shp)
    edg = jnp.stack([rowp, colp, wbits], axis=1)
    nu = user_emb.shape[0]
    ni = item_emb.shape[0]
    emb0 = jnp.concatenate(
        [user_emb, item_emb, jnp.zeros((NPAD - nu - ni, D), jnp.float32)],
        axis=0)
    emb0s = jnp.concatenate([emb0[:, :DH], emb0[:, DH:]], axis=0)
    emb1s = _layer(emb0s, edg)
    emb2s = _layer(emb1s, edg)
    fin = _mean(emb0, emb1s, emb2s)
    return fin[:nu], fin[nu:nu + ni]


# speedup vs baseline: 9.4133x; 1.0955x over previous
import dataclasses
import functools

import jax
import jax.numpy as jnp
from jax import lax
from jax.experimental import pallas as pl
from jax.experimental.pallas import tpu as pltpu
from jax.experimental.pallas import tpu_sc as plsc

D = 64
DH = 32
NPAD = 50176
STRIP = NPAD // 16
PIECE = 112
SUP = 28
SUB = 14
CHUNK = 128
EPAD = 16 * SUP * SUB * CHUNK
NRING = 5
LOOKAHEAD = 3


def _make_layer():
    scratch = (
        [pltpu.VMEM((3, SUB, CHUNK), jnp.int32)]
        + [pltpu.VMEM((CHUNK, DH), jnp.float32)
           for _ in range(NRING)]
        + [pltpu.VMEM((PIECE, DH), jnp.float32)]
        + [pltpu.VMEM_SHARED((NPAD, DH), jnp.float32)]
        + [pltpu.SemaphoreType.DMA for _ in range(2 * NRING)]
    )
    out_t = jax.ShapeDtypeStruct((2 * NPAD, DH), jnp.float32)
    mesh = plsc.VectorSubcoreMesh(core_axis_name="c", subcore_axis_name="s")

    def body(emb, edg, out, eb, *rest):
        bufs = rest[:NRING]
        piece = rest[NRING]
        acc = rest[NRING + 1]
        gsems = rest[NRING + 2:2 * NRING + 2]
        ssems = rest[2 * NRING + 2:]
        c = lax.axis_index("c")
        s = lax.axis_index("s")
        zero16 = jnp.zeros((16,), jnp.float32)
        tabbase = c * NPAD

        @pl.loop(0, PIECE)
        def _(i):
            for q in range(DH // 16):
                piece[i, pl.ds(q * 16, 16)] = zero16
        for p in range(STRIP // PIECE):
            pltpu.sync_copy(piece, acc.at[pl.ds(s * STRIP + p * PIECE, PIECE)])
        plsc.subcore_barrier()

        def scale(buf, jj):
            @pl.loop(0, CHUNK // 16)
            def _(g16):
                w16 = plsc.bitcast(eb[2, jj, pl.ds(g16 * 16, 16)], jnp.float32)
                base = g16 * 16
                for e in range(16):
                    wsp = jnp.full((16,), w16[e])
                    for q in range(DH // 16):
                        sl = pl.ds(q * 16, 16)
                        buf[base + e, sl] = buf[base + e, sl] * wsp

        @pl.loop(0, SUP)
        def _(g):
            pltpu.sync_copy(edg.at[s * SUP + g], eb)
            for jj in range(SUB):
                for k in range(CHUNK // 16):
                    sl = pl.ds(k * 16, 16)
                    eb[1, jj, sl] = eb[1, jj, sl] + tabbase
            gd = [None] * NRING
            sd = [None] * NRING

            def wait_sd(x):
                if sd[x] is not None:
                    sd[x].wait()
                    sd[x] = None

            def start_gather(jx):
                bx = jx % NRING
                wait_sd(bx)
                gd[bx] = pltpu.async_copy(
                    emb.at[eb.at[1].at[jx]], bufs[bx], gsems[bx])

            for jx in range(min(LOOKAHEAD, SUB)):
                start_gather(jx)
            for jj in range(SUB):
                b = jj % NRING
                gd[b].wait()
                scale(bufs[b], jj)
                sd[b] = pltpu.async_copy(bufs[b], acc.at[eb.at[0].at[jj]],
                                         ssems[b], add=True)
                if jj + LOOKAHEAD < SUB:
                    start_gather(jj + LOOKAHEAD)
            for x in range(NRING):
                wait_sd(x)
        plsc.subcore_barrier()

        for p in range(STRIP // PIECE):
            lo = s * STRIP + p * PIECE
            pltpu.sync_copy(acc.at[pl.ds(lo, PIECE)], piece)
            pltpu.sync_copy(piece, out.at[pl.ds(tabbase + lo, PIECE)])

    cp = pltpu.CompilerParams(use_tc_tiling_on_sc=False)
    if "needs_layout_passes" in pltpu.CompilerParams.__dataclass_fields__:
        cp = dataclasses.replace(cp, needs_layout_passes=False)
    return functools.partial(
        pl.kernel, out_type=out_t, mesh=mesh, scratch_types=scratch,
        compiler_params=cp)(body)


_layer = _make_layer()

_BM = 512


def _mean_body(e0, e1l, e1r, e2l, e2r, out):
    third = jnp.float32(1.0 / 3.0)
    out[:, :DH] = (e0[:, :DH] + e1l[...] + e2l[...]) * third
    out[:, DH:] = (e0[:, DH:] + e1r[...] + e2r[...]) * third


def _mean(e0, e1, e2):
    nb = NPAD // _BM
    half = lambda h: pl.BlockSpec((_BM, DH), lambda i, h=h: (i + h * nb, 0))
    return pl.pallas_call(
        _mean_body,
        grid=(nb,),
        in_specs=[pl.BlockSpec((_BM, D), lambda i: (i, 0)),
                  half(0), half(1), half(0), half(1)],
        out_specs=pl.BlockSpec((_BM, D), lambda i: (i, 0)),
        out_shape=jax.ShapeDtypeStruct((NPAD, D), jnp.float32),
    )(e0, e1, e1, e2, e2)


def kernel(edge_index, edge_weight, user_emb, item_emb):
    row = edge_index[0]
    col = edge_index[1]
    e = row.shape[0]
    padn = EPAD - e
    pad_i = jnp.zeros((padn,), jnp.int32)
    nsuper = 16 * SUP
    shp = (nsuper, SUB, CHUNK)
    rowp = jnp.concatenate([row, pad_i]).reshape(shp)
    colp = jnp.concatenate([col, pad_i]).reshape(shp)
    wbits = lax.bitcast_convert_type(
        jnp.concatenate([edge_weight, jnp.zeros((padn,), jnp.float32)]),
        jnp.int32).reshape(shp)
    edg = jnp.stack([rowp, colp, wbits], axis=1)
    nu = user_emb.shape[0]
    ni = item_emb.shape[0]
    emb0 = jnp.concatenate(
        [user_emb, item_emb, jnp.zeros((NPAD - nu - ni, D), jnp.float32)],
        axis=0)
    emb0s = jnp.concatenate([emb0[:, :DH], emb0[:, DH:]], axis=0)
    emb1s = _layer(emb0s, edg)
    emb2s = _layer(emb1s, edg)
    fin = _mean(emb0, emb1s, emb2s)
    return fin[:nu], fin[nu:nu + ni]
